# baseline (speedup 1.0000x reference)
import functools

import jax
import jax.numpy as jnp
from jax import lax
from jax.experimental import pallas as pl
from jax.experimental.pallas import tpu as pltpu
from jax.experimental.pallas import tpu_sc as plsc

B, N, D_IN = 16, 1024, 64
M = B * N
D_EMB = 32
V = 8192
TW = 128

TM = 128
SUB = 128
NSUB = V // SUB
GRID = M // TM
VC = 512
LOSS_SCALE = 2.0 / (M * D_EMB)


def _argmin_body(z_ref, wint_ref, cbt_ref, cb_ref, woutt_ref,
                 idx_ref, t_ref, loss_ref, cbt2_ref, cn_ref):
    i = pl.program_id(0)

    @pl.when(i == 0)
    def _():
        loss_ref[...] = jnp.zeros((1, 1), jnp.float32)
        def cbody(c, carry):
            rows = cb_ref[pl.ds(c * VC, VC), :]
            co = jnp.dot(rows, woutt_ref[...])
            t_ref[pl.ds(c * VC, VC), :] = jnp.concatenate(
                [co, jnp.zeros((VC, TW - D_IN), jnp.float32)], axis=1)
            return carry

        lax.fori_loop(0, V // VC, cbody, 0)
        cbt = cbt_ref[...]
        cbt2_ref[...] = cbt + cbt
        cn = jnp.sum(cbt * cbt, axis=0, keepdims=True)
        cn_ref[...] = jnp.broadcast_to(cn, (8, V))

    z = z_ref[...]
    z_in = jnp.dot(z, wint_ref[...])
    zn = jnp.sum(z_in * z_in, axis=1, keepdims=True)
    znb = jnp.broadcast_to(zn, (TM, SUB)).reshape(TM // 8, 8, SUB)

    def chunk_dot(j):
        j = jnp.minimum(j, NSUB - 1)
        return jnp.dot(z_in, cbt2_ref[:, pl.ds(j * SUB, SUB)])

    def body(j, carry):
        mv, mj, zc2 = carry
        zc2_next = chunk_dot(j + 1)
        cn8 = cn_ref[:, pl.ds(j * SUB, SUB)].reshape(1, 8, SUB)
        dist = (znb + cn8) - zc2.reshape(TM // 8, 8, SUB)
        better = dist < mv
        return (jnp.where(better, dist, mv),
                jnp.where(better, j, mj),
                zc2_next)

    mv0 = jnp.full((TM // 8, 8, SUB), jnp.inf, jnp.float32)
    mj0 = jnp.zeros((TM // 8, 8, SUB), jnp.int32)
    mv3, mj3, _ = lax.fori_loop(0, NSUB, body, (mv0, mj0, chunk_dot(0)),
                                unroll=1)
    mv, mj = mv3.reshape(TM, SUB), mj3.reshape(TM, SUB)

    lv = jnp.min(mv, axis=1, keepdims=True)
    lanes = lax.broadcasted_iota(jnp.int32, (TM, SUB), 1)
    cand = jnp.where(mv == lv, mj * SUB + lanes, 2**31 - 1)
    idx_ref[...] = jnp.min(cand, axis=1, keepdims=True)
    loss_ref[...] += jnp.sum(lv).reshape(1, 1)

    @pl.when(i == pl.num_programs(0) - 1)
    def _():
        loss_ref[...] *= LOSS_SCALE


def _run_argmin(zf, wint, cbt, cb, woutt):
    return pl.pallas_call(
        _argmin_body,
        grid=(GRID,),
        in_specs=[
            pl.BlockSpec((TM, D_IN), lambda i: (i, 0)),
            pl.BlockSpec((D_IN, D_EMB), lambda i: (0, 0)),
            pl.BlockSpec((D_EMB, V), lambda i: (0, 0)),
            pl.BlockSpec((V, D_EMB), lambda i: (0, 0)),
            pl.BlockSpec((D_EMB, D_IN), lambda i: (0, 0)),
        ],
        out_specs=[
            pl.BlockSpec((TM, 1), lambda i: (i, 0)),
            pl.BlockSpec((V, TW), lambda i: (0, 0)),
            pl.BlockSpec((1, 1), lambda i: (0, 0)),
        ],
        out_shape=[
            jax.ShapeDtypeStruct((M, 1), jnp.int32),
            jax.ShapeDtypeStruct((V, TW), jnp.float32),
            jax.ShapeDtypeStruct((1, 1), jnp.float32),
        ],
        scratch_shapes=[
            pltpu.VMEM((D_EMB, V), jnp.float32),
            pltpu.VMEM((8, V), jnp.float32),
        ],
    )(zf, wint, cbt, cb, woutt)


NC, NS = 2, 16
NW = NC * NS
B_PER_W = M // NW
KCH = B_PER_W // 128


def _gather_body(table_hbm, idx_hbm, out_hbm, idx_v, rows_v, sem):
    wid = lax.axis_index("s") * NC + lax.axis_index("c")
    pltpu.sync_copy(idx_hbm.at[wid], idx_v)
    copies = [
        pltpu.async_copy(table_hbm.at[idx_v.at[j]], rows_v.at[j], sem)
        for j in range(KCH)
    ]
    for cp in copies:
        cp.wait()
    pltpu.sync_copy(rows_v, out_hbm.at[wid])


def _run_gather(table, idx3d):
    mesh = plsc.VectorSubcoreMesh(core_axis_name="c", subcore_axis_name="s")
    k = functools.partial(
        pl.kernel,
        mesh=mesh,
        out_type=jax.ShapeDtypeStruct((NW, KCH, 128, TW), jnp.float32),
        scratch_types=[
            pltpu.VMEM((KCH, 128), jnp.int32),
            pltpu.VMEM((KCH, 128, TW), jnp.float32),
            pltpu.SemaphoreType.DMA,
        ],
    )(_gather_body)
    return k(table, idx3d)


def kernel(z, codebook, W_in, W_out):
    zf = z.reshape(M, D_IN)
    idx2, table, loss = _run_argmin(zf, W_in.T, codebook.T, codebook, W_out.T)
    idx3d = idx2.reshape(NW, KCH, 128)
    rows = _run_gather(table, idx3d).reshape(M, TW)
    out = rows[:, :D_IN]
    return (out.reshape(B, N, D_IN),
            idx2.reshape(B, N),
            loss[0, 0])


# speedup vs baseline: 3.3231x; 3.3231x over previous
import functools

import jax
import jax.numpy as jnp
from jax import lax
from jax.experimental import pallas as pl
from jax.experimental.pallas import tpu as pltpu
from jax.experimental.pallas import tpu_sc as plsc

B, N, D_IN = 16, 1024, 64
M = B * N
D_EMB = 32
V = 8192
TW = 128

TM = 128
SUB = 128
NSUB = V // SUB
GRID = M // TM
VC = 512
LOSS_SCALE = 2.0 / (M * D_EMB)


def _argmin_body(z_ref, wint_ref, cbt_ref, cb_ref, woutt_ref,
                 idx_ref, t_ref, loss_ref, cbt2_ref, cn_ref):
    i = pl.program_id(0)

    @pl.when(i == 0)
    def _():
        loss_ref[...] = jnp.zeros((1, 1), jnp.float32)
        def cbody(c, carry):
            rows = cb_ref[pl.ds(c * VC, VC), :]
            co = jnp.dot(rows, woutt_ref[...])
            t_ref[pl.ds(c * VC, VC), :] = jnp.concatenate(
                [co, jnp.zeros((VC, TW - D_IN), jnp.float32)], axis=1)
            return carry

        lax.fori_loop(0, V // VC, cbody, 0)
        cbt = cbt_ref[...]
        cbt2_ref[...] = cbt + cbt
        cn = jnp.sum(cbt * cbt, axis=0, keepdims=True)
        cn_ref[...] = jnp.broadcast_to(cn, (8, V))

    z = z_ref[...]
    z_in = jnp.dot(z, wint_ref[...])
    zn = jnp.sum(z_in * z_in, axis=1, keepdims=True)
    znb = jnp.broadcast_to(zn, (TM, SUB)).reshape(TM // 8, 8, SUB)

    def body(j, carry):
        mv, mj = carry
        cbt2 = cbt2_ref[:, pl.ds(j * SUB, SUB)]
        cn8 = cn_ref[:, pl.ds(j * SUB, SUB)].reshape(1, 8, SUB)
        zc2 = jnp.dot(z_in, cbt2).reshape(TM // 8, 8, SUB)
        dist = (znb + cn8) - zc2
        better = dist < mv
        return (jnp.where(better, dist, mv),
                jnp.where(better, j, mj))

    mv0 = jnp.full((TM // 8, 8, SUB), jnp.inf, jnp.float32)
    mj0 = jnp.zeros((TM // 8, 8, SUB), jnp.int32)
    mv3, mj3 = lax.fori_loop(0, NSUB, body, (mv0, mj0), unroll=8)
    mv, mj = mv3.reshape(TM, SUB), mj3.reshape(TM, SUB)

    lv = jnp.min(mv, axis=1, keepdims=True)
    lanes = lax.broadcasted_iota(jnp.int32, (TM, SUB), 1)
    cand = jnp.where(mv == lv, mj * SUB + lanes, 2**31 - 1)
    idx_ref[...] = jnp.min(cand, axis=1, keepdims=True)
    loss_ref[...] += jnp.sum(lv).reshape(1, 1)

    @pl.when(i == pl.num_programs(0) - 1)
    def _():
        loss_ref[...] *= LOSS_SCALE


def _run_argmin(zf, wint, cbt, cb, woutt):
    return pl.pallas_call(
        _argmin_body,
        grid=(GRID,),
        in_specs=[
            pl.BlockSpec((TM, D_IN), lambda i: (i, 0)),
            pl.BlockSpec((D_IN, D_EMB), lambda i: (0, 0)),
            pl.BlockSpec((D_EMB, V), lambda i: (0, 0)),
            pl.BlockSpec((V, D_EMB), lambda i: (0, 0)),
            pl.BlockSpec((D_EMB, D_IN), lambda i: (0, 0)),
        ],
        out_specs=[
            pl.BlockSpec((TM, 1), lambda i: (i, 0)),
            pl.BlockSpec((V, TW), lambda i: (0, 0)),
            pl.BlockSpec((1, 1), lambda i: (0, 0)),
        ],
        out_shape=[
            jax.ShapeDtypeStruct((M, 1), jnp.int32),
            jax.ShapeDtypeStruct((V, TW), jnp.float32),
            jax.ShapeDtypeStruct((1, 1), jnp.float32),
        ],
        scratch_shapes=[
            pltpu.VMEM((D_EMB, V), jnp.float32),
            pltpu.VMEM((8, V), jnp.float32),
        ],
    )(zf, wint, cbt, cb, woutt)


NC, NS = 2, 16
NW = NC * NS
B_PER_W = M // NW
KCH = B_PER_W // 128


def _gather_body(table_hbm, idx_hbm, out_hbm, idx_v, rows_v, sem):
    wid = lax.axis_index("s") * NC + lax.axis_index("c")
    pltpu.sync_copy(idx_hbm.at[wid], idx_v)
    copies = [
        pltpu.async_copy(table_hbm.at[idx_v.at[j]], rows_v.at[j], sem)
        for j in range(KCH)
    ]
    for cp in copies:
        cp.wait()
    pltpu.sync_copy(rows_v, out_hbm.at[wid])


def _run_gather(table, idx3d):
    mesh = plsc.VectorSubcoreMesh(core_axis_name="c", subcore_axis_name="s")
    k = functools.partial(
        pl.kernel,
        mesh=mesh,
        out_type=jax.ShapeDtypeStruct((NW, KCH, 128, TW), jnp.float32),
        scratch_types=[
            pltpu.VMEM((KCH, 128), jnp.int32),
            pltpu.VMEM((KCH, 128, TW), jnp.float32),
            pltpu.SemaphoreType.DMA,
        ],
    )(_gather_body)
    return k(table, idx3d)


def kernel(z, codebook, W_in, W_out):
    zf = z.reshape(M, D_IN)
    idx2, table, loss = _run_argmin(zf, W_in.T, codebook.T, codebook, W_out.T)
    idx3d = idx2.reshape(NW, KCH, 128)
    rows = _run_gather(table, idx3d).reshape(M, TW)
    out = rows[:, :D_IN]
    return (out.reshape(B, N, D_IN),
            idx2.reshape(B, N),
            loss[0, 0])


# speedup vs baseline: 4.1408x; 1.2461x over previous
import functools

import jax
import jax.numpy as jnp
from jax import lax
from jax.experimental import pallas as pl
from jax.experimental.pallas import tpu as pltpu
from jax.experimental.pallas import tpu_sc as plsc

B, N, D_IN = 16, 1024, 64
M = B * N
D_EMB = 32
V = 8192
TW = 128

TM = 128
SUB = 128
NSUB = V // SUB
GRID = M // TM
VC = 512
LOSS_SCALE = 2.0 / (M * D_EMB)


def _argmin_body(z_ref, wint_ref, cbt_ref, cb_ref, woutt_ref,
                 idx_ref, t_ref, loss_ref, cbt2_ref, cn_ref):
    i = pl.program_id(0)

    @pl.when(i == 0)
    def _():
        loss_ref[...] = jnp.zeros((1, 1), jnp.float32)
        def cbody(c, carry):
            rows = cb_ref[pl.ds(c * VC, VC), :]
            co = jnp.dot(rows, woutt_ref[...])
            t_ref[pl.ds(c * VC, VC), :] = jnp.concatenate(
                [co, jnp.zeros((VC, TW - D_IN), jnp.float32)], axis=1)
            return carry

        lax.fori_loop(0, V // VC, cbody, 0)
        cbt = cbt_ref[...]
        cbt2_ref[...] = cbt + cbt
        cn = jnp.sum(cbt * cbt, axis=0, keepdims=True)
        cn_ref[...] = jnp.broadcast_to(cn, (8, V))

    z = z_ref[...]
    z_in = jnp.dot(z, wint_ref[...])
    zn = jnp.sum(z_in * z_in, axis=1, keepdims=True)
    znb = jnp.broadcast_to(zn, (TM, SUB)).reshape(TM // 8, 8, SUB)

    def body(j, carry):
        mv, mj = carry
        cbt2 = cbt2_ref[:, pl.ds(j * SUB, SUB)]
        cn8 = cn_ref[:, pl.ds(j * SUB, SUB)].reshape(1, 8, SUB)
        zc2 = jnp.dot(z_in, cbt2).reshape(TM // 8, 8, SUB)
        dist = (znb + cn8) - zc2
        better = dist < mv
        return (jnp.where(better, dist, mv),
                jnp.where(better, j, mj))

    mv0 = jnp.full((TM // 8, 8, SUB), jnp.inf, jnp.float32)
    mj0 = jnp.zeros((TM // 8, 8, SUB), jnp.int32)
    mv3, mj3 = lax.fori_loop(0, NSUB, body, (mv0, mj0), unroll=16)
    mv, mj = mv3.reshape(TM, SUB), mj3.reshape(TM, SUB)

    lv = jnp.min(mv, axis=1, keepdims=True)
    lanes = lax.broadcasted_iota(jnp.int32, (TM, SUB), 1)
    cand = jnp.where(mv == lv, mj * SUB + lanes, 2**31 - 1)
    idx_ref[...] = jnp.min(cand, axis=1, keepdims=True)
    loss_ref[...] += jnp.sum(lv).reshape(1, 1)

    @pl.when(i == pl.num_programs(0) - 1)
    def _():
        loss_ref[...] *= LOSS_SCALE


def _run_argmin(zf, wint, cbt, cb, woutt):
    return pl.pallas_call(
        _argmin_body,
        grid=(GRID,),
        in_specs=[
            pl.BlockSpec((TM, D_IN), lambda i: (i, 0)),
            pl.BlockSpec((D_IN, D_EMB), lambda i: (0, 0)),
            pl.BlockSpec((D_EMB, V), lambda i: (0, 0)),
            pl.BlockSpec((V, D_EMB), lambda i: (0, 0)),
            pl.BlockSpec((D_EMB, D_IN), lambda i: (0, 0)),
        ],
        out_specs=[
            pl.BlockSpec((TM, 1), lambda i: (i, 0)),
            pl.BlockSpec((V, TW), lambda i: (0, 0)),
            pl.BlockSpec((1, 1), lambda i: (0, 0)),
        ],
        out_shape=[
            jax.ShapeDtypeStruct((M, 1), jnp.int32),
            jax.ShapeDtypeStruct((V, TW), jnp.float32),
            jax.ShapeDtypeStruct((1, 1), jnp.float32),
        ],
        scratch_shapes=[
            pltpu.VMEM((D_EMB, V), jnp.float32),
            pltpu.VMEM((8, V), jnp.float32),
        ],
    )(zf, wint, cbt, cb, woutt)


NC, NS = 2, 16
NW = NC * NS
B_PER_W = M // NW
KCH = B_PER_W // 128


def _gather_body(table_hbm, idx_hbm, out_hbm, idx_v, rows_v, sem):
    wid = lax.axis_index("s") * NC + lax.axis_index("c")
    pltpu.sync_copy(idx_hbm.at[wid], idx_v)
    copies = [
        pltpu.async_copy(table_hbm.at[idx_v.at[j]], rows_v.at[j], sem)
        for j in range(KCH)
    ]
    for cp in copies:
        cp.wait()
    pltpu.sync_copy(rows_v, out_hbm.at[wid])


def _run_gather(table, idx3d):
    mesh = plsc.VectorSubcoreMesh(core_axis_name="c", subcore_axis_name="s")
    k = functools.partial(
        pl.kernel,
        mesh=mesh,
        out_type=jax.ShapeDtypeStruct((NW, KCH, 128, TW), jnp.float32),
        scratch_types=[
            pltpu.VMEM((KCH, 128), jnp.int32),
            pltpu.VMEM((KCH, 128, TW), jnp.float32),
            pltpu.SemaphoreType.DMA,
        ],
    )(_gather_body)
    return k(table, idx3d)


def kernel(z, codebook, W_in, W_out):
    zf = z.reshape(M, D_IN)
    idx2, table, loss = _run_argmin(zf, W_in.T, codebook.T, codebook, W_out.T)
    idx3d = idx2.reshape(NW, KCH, 128)
    rows = _run_gather(table, idx3d).reshape(M, TW)
    out = rows[:, :D_IN]
    return (out.reshape(B, N, D_IN),
            idx2.reshape(B, N),
            loss[0, 0])


# speedup vs baseline: 4.6222x; 1.1163x over previous
import functools

import jax
import jax.numpy as jnp
from jax import lax
from jax.experimental import pallas as pl
from jax.experimental.pallas import tpu as pltpu
from jax.experimental.pallas import tpu_sc as plsc

B, N, D_IN = 16, 1024, 64
M = B * N
D_EMB = 32
V = 8192
TW = 128

TM = 128
SUB = 128
NSUB = V // SUB
GRID = M // TM
VC = 512
LOSS_SCALE = 2.0 / (M * D_EMB)


def _argmin_body(z_ref, wint_ref, cbt_ref, cb_ref, woutt_ref,
                 idx_ref, t_ref, loss_ref, cbt2_ref, cn_ref):
    i = pl.program_id(0)

    @pl.when(i == 0)
    def _():
        loss_ref[...] = jnp.zeros((1, 1), jnp.float32)
        def cbody(c, carry):
            rows = cb_ref[pl.ds(c * VC, VC), :]
            co = jnp.dot(rows, woutt_ref[...])
            t_ref[pl.ds(c * VC, VC), :] = jnp.concatenate(
                [co, jnp.zeros((VC, TW - D_IN), jnp.float32)], axis=1)
            return carry

        lax.fori_loop(0, V // VC, cbody, 0)
        cbt = cbt_ref[...]
        cbt2_ref[...] = cbt + cbt
        cn = jnp.sum(cbt * cbt, axis=0, keepdims=True)
        cn_ref[...] = jnp.broadcast_to(cn, (8, V))

    z = z_ref[...]
    z_in = jnp.dot(z, wint_ref[...])
    zn = jnp.sum(z_in * z_in, axis=1, keepdims=True)
    znb = jnp.broadcast_to(zn, (TM, SUB)).reshape(TM // 8, 8, SUB)

    def body(j, carry):
        mv, mj = carry
        cbt2 = cbt2_ref[:, pl.ds(j * SUB, SUB)]
        cn8 = cn_ref[:, pl.ds(j * SUB, SUB)].reshape(1, 8, SUB)
        zc2 = jnp.dot(z_in, cbt2).reshape(TM // 8, 8, SUB)
        dist = (znb + cn8) - zc2
        better = dist < mv
        return (jnp.where(better, dist, mv),
                jnp.where(better, j, mj))

    mv0 = jnp.full((TM // 8, 8, SUB), jnp.inf, jnp.float32)
    mj0 = jnp.zeros((TM // 8, 8, SUB), jnp.int32)
    mv3, mj3 = lax.fori_loop(0, NSUB, body, (mv0, mj0), unroll=32)
    mv, mj = mv3.reshape(TM, SUB), mj3.reshape(TM, SUB)

    lv = jnp.min(mv, axis=1, keepdims=True)
    lanes = lax.broadcasted_iota(jnp.int32, (TM, SUB), 1)
    cand = jnp.where(mv == lv, mj * SUB + lanes, 2**31 - 1)
    idx_ref[...] = jnp.min(cand, axis=1, keepdims=True)
    loss_ref[...] += jnp.sum(lv).reshape(1, 1)

    @pl.when(i == pl.num_programs(0) - 1)
    def _():
        loss_ref[...] *= LOSS_SCALE


def _run_argmin(zf, wint, cbt, cb, woutt):
    return pl.pallas_call(
        _argmin_body,
        grid=(GRID,),
        in_specs=[
            pl.BlockSpec((TM, D_IN), lambda i: (i, 0)),
            pl.BlockSpec((D_IN, D_EMB), lambda i: (0, 0)),
            pl.BlockSpec((D_EMB, V), lambda i: (0, 0)),
            pl.BlockSpec((V, D_EMB), lambda i: (0, 0)),
            pl.BlockSpec((D_EMB, D_IN), lambda i: (0, 0)),
        ],
        out_specs=[
            pl.BlockSpec((TM, 1), lambda i: (i, 0)),
            pl.BlockSpec((V, TW), lambda i: (0, 0)),
            pl.BlockSpec((1, 1), lambda i: (0, 0)),
        ],
        out_shape=[
            jax.ShapeDtypeStruct((M, 1), jnp.int32),
            jax.ShapeDtypeStruct((V, TW), jnp.float32),
            jax.ShapeDtypeStruct((1, 1), jnp.float32),
        ],
        scratch_shapes=[
            pltpu.VMEM((D_EMB, V), jnp.float32),
            pltpu.VMEM((8, V), jnp.float32),
        ],
    )(zf, wint, cbt, cb, woutt)


NC, NS = 2, 16
NW = NC * NS
B_PER_W = M // NW
KCH = B_PER_W // 128


def _gather_body(table_hbm, idx_hbm, out_hbm, idx_v, rows_v, sem):
    wid = lax.axis_index("s") * NC + lax.axis_index("c")
    pltpu.sync_copy(idx_hbm.at[wid], idx_v)
    copies = [
        pltpu.async_copy(table_hbm.at[idx_v.at[j]], rows_v.at[j], sem)
        for j in range(KCH)
    ]
    for cp in copies:
        cp.wait()
    pltpu.sync_copy(rows_v, out_hbm.at[wid])


def _run_gather(table, idx3d):
    mesh = plsc.VectorSubcoreMesh(core_axis_name="c", subcore_axis_name="s")
    k = functools.partial(
        pl.kernel,
        mesh=mesh,
        out_type=jax.ShapeDtypeStruct((NW, KCH, 128, TW), jnp.float32),
        scratch_types=[
            pltpu.VMEM((KCH, 128), jnp.int32),
            pltpu.VMEM((KCH, 128, TW), jnp.float32),
            pltpu.SemaphoreType.DMA,
        ],
    )(_gather_body)
    return k(table, idx3d)


def kernel(z, codebook, W_in, W_out):
    zf = z.reshape(M, D_IN)
    idx2, table, loss = _run_argmin(zf, W_in.T, codebook.T, codebook, W_out.T)
    idx3d = idx2.reshape(NW, KCH, 128)
    rows = _run_gather(table, idx3d).reshape(M, TW)
    out = rows[:, :D_IN]
    return (out.reshape(B, N, D_IN),
            idx2.reshape(B, N),
            loss[0, 0])


# speedup vs baseline: 5.4322x; 1.1753x over previous
import functools

import jax
import jax.numpy as jnp
from jax import lax
from jax.experimental import pallas as pl
from jax.experimental.pallas import tpu as pltpu
from jax.experimental.pallas import tpu_sc as plsc

B, N, D_IN = 16, 1024, 64
M = B * N
D_EMB = 32
V = 8192
TW = 128

TM = 128
SUB = 128
NSUB = V // SUB
GRID = M // TM
VC = 512
LOSS_SCALE = 2.0 / (M * D_EMB)


def _argmin_body(z_ref, wint_ref, cbt_ref, cb_ref, woutt_ref,
                 idx_ref, t_ref, loss_ref, cbt2_ref, cn_ref):
    i = pl.program_id(0)

    @pl.when(i == 0)
    def _():
        loss_ref[...] = jnp.zeros((1, 1), jnp.float32)
        def cbody(c, carry):
            rows = cb_ref[pl.ds(c * VC, VC), :]
            co = jnp.dot(rows, woutt_ref[...])
            t_ref[pl.ds(c * VC, VC), :] = jnp.concatenate(
                [co, jnp.zeros((VC, TW - D_IN), jnp.float32)], axis=1)
            return carry

        lax.fori_loop(0, V // VC, cbody, 0)
        cbt = cbt_ref[...]
        cbt2_ref[...] = cbt + cbt
        cn = jnp.sum(cbt * cbt, axis=0, keepdims=True)
        cn_ref[...] = jnp.broadcast_to(cn, (8, V))

    z = z_ref[...]
    z_in = jnp.dot(z, wint_ref[...])
    zn = jnp.sum(z_in * z_in, axis=1, keepdims=True)
    znb = jnp.broadcast_to(zn, (TM, SUB)).reshape(TM // 8, 8, SUB)

    def body(j, carry):
        mv, mj = carry
        cbt2 = cbt2_ref[:, pl.ds(j * SUB, SUB)]
        cn8 = cn_ref[:, pl.ds(j * SUB, SUB)].reshape(1, 8, SUB)
        zc2 = jnp.dot(z_in, cbt2).reshape(TM // 8, 8, SUB)
        dist = (znb + cn8) - zc2
        better = dist < mv
        return (jnp.where(better, dist, mv),
                jnp.where(better, j, mj))

    mv0 = jnp.full((TM // 8, 8, SUB), jnp.inf, jnp.float32)
    mj0 = jnp.zeros((TM // 8, 8, SUB), jnp.int32)
    mv3, mj3 = lax.fori_loop(0, NSUB, body, (mv0, mj0), unroll=64)
    mv, mj = mv3.reshape(TM, SUB), mj3.reshape(TM, SUB)

    lv = jnp.min(mv, axis=1, keepdims=True)
    lanes = lax.broadcasted_iota(jnp.int32, (TM, SUB), 1)
    cand = jnp.where(mv == lv, mj * SUB + lanes, 2**31 - 1)
    idx_ref[...] = jnp.min(cand, axis=1, keepdims=True)
    loss_ref[...] += jnp.sum(lv).reshape(1, 1)

    @pl.when(i == pl.num_programs(0) - 1)
    def _():
        loss_ref[...] *= LOSS_SCALE


def _run_argmin(zf, wint, cbt, cb, woutt):
    return pl.pallas_call(
        _argmin_body,
        grid=(GRID,),
        in_specs=[
            pl.BlockSpec((TM, D_IN), lambda i: (i, 0)),
            pl.BlockSpec((D_IN, D_EMB), lambda i: (0, 0)),
            pl.BlockSpec((D_EMB, V), lambda i: (0, 0)),
            pl.BlockSpec((V, D_EMB), lambda i: (0, 0)),
            pl.BlockSpec((D_EMB, D_IN), lambda i: (0, 0)),
        ],
        out_specs=[
            pl.BlockSpec((TM, 1), lambda i: (i, 0)),
            pl.BlockSpec((V, TW), lambda i: (0, 0)),
            pl.BlockSpec((1, 1), lambda i: (0, 0)),
        ],
        out_shape=[
            jax.ShapeDtypeStruct((M, 1), jnp.int32),
            jax.ShapeDtypeStruct((V, TW), jnp.float32),
            jax.ShapeDtypeStruct((1, 1), jnp.float32),
        ],
        scratch_shapes=[
            pltpu.VMEM((D_EMB, V), jnp.float32),
            pltpu.VMEM((8, V), jnp.float32),
        ],
    )(zf, wint, cbt, cb, woutt)


NC, NS = 2, 16
NW = NC * NS
B_PER_W = M // NW
KCH = B_PER_W // 128


def _gather_body(table_hbm, idx_hbm, out_hbm, idx_v, rows_v, sem):
    wid = lax.axis_index("s") * NC + lax.axis_index("c")
    pltpu.sync_copy(idx_hbm.at[wid], idx_v)
    copies = [
        pltpu.async_copy(table_hbm.at[idx_v.at[j]], rows_v.at[j], sem)
        for j in range(KCH)
    ]
    for cp in copies:
        cp.wait()
    pltpu.sync_copy(rows_v, out_hbm.at[wid])


def _run_gather(table, idx3d):
    mesh = plsc.VectorSubcoreMesh(core_axis_name="c", subcore_axis_name="s")
    k = functools.partial(
        pl.kernel,
        mesh=mesh,
        out_type=jax.ShapeDtypeStruct((NW, KCH, 128, TW), jnp.float32),
        scratch_types=[
            pltpu.VMEM((KCH, 128), jnp.int32),
            pltpu.VMEM((KCH, 128, TW), jnp.float32),
            pltpu.SemaphoreType.DMA,
        ],
    )(_gather_body)
    return k(table, idx3d)


def kernel(z, codebook, W_in, W_out):
    zf = z.reshape(M, D_IN)
    idx2, table, loss = _run_argmin(zf, W_in.T, codebook.T, codebook, W_out.T)
    idx3d = idx2.reshape(NW, KCH, 128)
    rows = _run_gather(table, idx3d).reshape(M, TW)
    out = rows[:, :D_IN]
    return (out.reshape(B, N, D_IN),
            idx2.reshape(B, N),
            loss[0, 0])


# speedup vs baseline: 6.0201x; 1.1082x over previous
import functools

import jax
import jax.numpy as jnp
from jax import lax
from jax.experimental import pallas as pl
from jax.experimental.pallas import tpu as pltpu
from jax.experimental.pallas import tpu_sc as plsc

B, N, D_IN = 16, 1024, 64
M = B * N
D_EMB = 32
V = 8192
TW = 128

TM = 256
SUB = 128
NSUB = V // SUB
GRID = M // TM
VC = 512
LOSS_SCALE = 2.0 / (M * D_EMB)


def _argmin_body(z_ref, wint_ref, cbt_ref, cb_ref, woutt_ref,
                 idx_ref, t_ref, loss_ref, cbt2_ref, cn_ref):
    i = pl.program_id(0)

    @pl.when(i == 0)
    def _():
        loss_ref[...] = jnp.zeros((1, 1), jnp.float32)
        def cbody(c, carry):
            rows = cb_ref[pl.ds(c * VC, VC), :]
            co = jnp.dot(rows, woutt_ref[...])
            t_ref[pl.ds(c * VC, VC), :] = jnp.concatenate(
                [co, jnp.zeros((VC, TW - D_IN), jnp.float32)], axis=1)
            return carry

        lax.fori_loop(0, V // VC, cbody, 0)
        cbt = cbt_ref[...]
        cbt2_ref[...] = cbt + cbt
        cn = jnp.sum(cbt * cbt, axis=0, keepdims=True)
        cn_ref[...] = jnp.broadcast_to(cn, (8, V))

    z = z_ref[...]
    z_in = jnp.dot(z, wint_ref[...])
    zn = jnp.sum(z_in * z_in, axis=1, keepdims=True)
    znb = jnp.broadcast_to(zn, (TM, SUB)).reshape(TM // 8, 8, SUB)

    def body(j, carry):
        mv, mj = carry
        cbt2 = cbt2_ref[:, pl.ds(j * SUB, SUB)]
        cn8 = cn_ref[:, pl.ds(j * SUB, SUB)].reshape(1, 8, SUB)
        zc2 = jnp.dot(z_in, cbt2).reshape(TM // 8, 8, SUB)
        dist = (znb + cn8) - zc2
        better = dist < mv
        return (jnp.where(better, dist, mv),
                jnp.where(better, j, mj))

    mv0 = jnp.full((TM // 8, 8, SUB), jnp.inf, jnp.float32)
    mj0 = jnp.zeros((TM // 8, 8, SUB), jnp.int32)
    mv3, mj3 = lax.fori_loop(0, NSUB, body, (mv0, mj0), unroll=64)
    mv, mj = mv3.reshape(TM, SUB), mj3.reshape(TM, SUB)

    lv = jnp.min(mv, axis=1, keepdims=True)
    lanes = lax.broadcasted_iota(jnp.int32, (TM, SUB), 1)
    cand = jnp.where(mv == lv, mj * SUB + lanes, 2**31 - 1)
    idx_ref[...] = jnp.min(cand, axis=1, keepdims=True)
    loss_ref[...] += jnp.sum(lv).reshape(1, 1)

    @pl.when(i == pl.num_programs(0) - 1)
    def _():
        loss_ref[...] *= LOSS_SCALE


def _run_argmin(zf, wint, cbt, cb, woutt):
    return pl.pallas_call(
        _argmin_body,
        grid=(GRID,),
        in_specs=[
            pl.BlockSpec((TM, D_IN), lambda i: (i, 0)),
            pl.BlockSpec((D_IN, D_EMB), lambda i: (0, 0)),
            pl.BlockSpec((D_EMB, V), lambda i: (0, 0)),
            pl.BlockSpec((V, D_EMB), lambda i: (0, 0)),
            pl.BlockSpec((D_EMB, D_IN), lambda i: (0, 0)),
        ],
        out_specs=[
            pl.BlockSpec((TM, 1), lambda i: (i, 0)),
            pl.BlockSpec((V, TW), lambda i: (0, 0)),
            pl.BlockSpec((1, 1), lambda i: (0, 0)),
        ],
        out_shape=[
            jax.ShapeDtypeStruct((M, 1), jnp.int32),
            jax.ShapeDtypeStruct((V, TW), jnp.float32),
            jax.ShapeDtypeStruct((1, 1), jnp.float32),
        ],
        scratch_shapes=[
            pltpu.VMEM((D_EMB, V), jnp.float32),
            pltpu.VMEM((8, V), jnp.float32),
        ],
    )(zf, wint, cbt, cb, woutt)


NC, NS = 2, 16
NW = NC * NS
B_PER_W = M // NW
KCH = B_PER_W // 128


def _gather_body(table_hbm, idx_hbm, out_hbm, idx_v, rows_v, sem):
    wid = lax.axis_index("s") * NC + lax.axis_index("c")
    pltpu.sync_copy(idx_hbm.at[wid], idx_v)
    copies = [
        pltpu.async_copy(table_hbm.at[idx_v.at[j]], rows_v.at[j], sem)
        for j in range(KCH)
    ]
    for cp in copies:
        cp.wait()
    pltpu.sync_copy(rows_v, out_hbm.at[wid])


def _run_gather(table, idx3d):
    mesh = plsc.VectorSubcoreMesh(core_axis_name="c", subcore_axis_name="s")
    k = functools.partial(
        pl.kernel,
        mesh=mesh,
        out_type=jax.ShapeDtypeStruct((NW, KCH, 128, TW), jnp.float32),
        scratch_types=[
            pltpu.VMEM((KCH, 128), jnp.int32),
            pltpu.VMEM((KCH, 128, TW), jnp.float32),
            pltpu.SemaphoreType.DMA,
        ],
    )(_gather_body)
    return k(table, idx3d)


def kernel(z, codebook, W_in, W_out):
    zf = z.reshape(M, D_IN)
    idx2, table, loss = _run_argmin(zf, W_in.T, codebook.T, codebook, W_out.T)
    idx3d = idx2.reshape(NW, KCH, 128)
    rows = _run_gather(table, idx3d).reshape(M, TW)
    out = rows[:, :D_IN]
    return (out.reshape(B, N, D_IN),
            idx2.reshape(B, N),
            loss[0, 0])


# speedup vs baseline: 6.2774x; 1.0428x over previous
import functools

import jax
import jax.numpy as jnp
from jax import lax
from jax.experimental import pallas as pl
from jax.experimental.pallas import tpu as pltpu
from jax.experimental.pallas import tpu_sc as plsc

B, N, D_IN = 16, 1024, 64
M = B * N
D_EMB = 32
V = 8192
TW = 128

TM = 512
SUB = 128
NSUB = V // SUB
GRID = M // TM
VC = 512
LOSS_SCALE = 2.0 / (M * D_EMB)


def _argmin_body(z_ref, wint_ref, cbt_ref, cb_ref, woutt_ref,
                 idx_ref, t_ref, loss_ref, cbt2_ref, cn_ref):
    i = pl.program_id(0)

    @pl.when(i == 0)
    def _():
        loss_ref[...] = jnp.zeros((1, 1), jnp.float32)
        def cbody(c, carry):
            rows = cb_ref[pl.ds(c * VC, VC), :]
            co = jnp.dot(rows, woutt_ref[...])
            t_ref[pl.ds(c * VC, VC), :] = jnp.concatenate(
                [co, jnp.zeros((VC, TW - D_IN), jnp.float32)], axis=1)
            return carry

        lax.fori_loop(0, V // VC, cbody, 0)
        cbt = cbt_ref[...]
        cbt2_ref[...] = cbt + cbt
        cn = jnp.sum(cbt * cbt, axis=0, keepdims=True)
        cn_ref[...] = jnp.broadcast_to(cn, (8, V))

    z = z_ref[...]
    z_in = jnp.dot(z, wint_ref[...])
    zn = jnp.sum(z_in * z_in, axis=1, keepdims=True)
    znb = jnp.broadcast_to(zn, (TM, SUB)).reshape(TM // 8, 8, SUB)

    def body(j, carry):
        mv, mj = carry
        cbt2 = cbt2_ref[:, pl.ds(j * SUB, SUB)]
        cn8 = cn_ref[:, pl.ds(j * SUB, SUB)].reshape(1, 8, SUB)
        zc2 = jnp.dot(z_in, cbt2).reshape(TM // 8, 8, SUB)
        dist = (znb + cn8) - zc2
        better = dist < mv
        return (jnp.where(better, dist, mv),
                jnp.where(better, j, mj))

    mv0 = jnp.full((TM // 8, 8, SUB), jnp.inf, jnp.float32)
    mj0 = jnp.zeros((TM // 8, 8, SUB), jnp.int32)
    mv3, mj3 = lax.fori_loop(0, NSUB, body, (mv0, mj0), unroll=64)
    mv, mj = mv3.reshape(TM, SUB), mj3.reshape(TM, SUB)

    lv = jnp.min(mv, axis=1, keepdims=True)
    lanes = lax.broadcasted_iota(jnp.int32, (TM, SUB), 1)
    cand = jnp.where(mv == lv, mj * SUB + lanes, 2**31 - 1)
    idx_ref[...] = jnp.min(cand, axis=1, keepdims=True)
    loss_ref[...] += jnp.sum(lv).reshape(1, 1)

    @pl.when(i == pl.num_programs(0) - 1)
    def _():
        loss_ref[...] *= LOSS_SCALE


def _run_argmin(zf, wint, cbt, cb, woutt):
    return pl.pallas_call(
        _argmin_body,
        grid=(GRID,),
        in_specs=[
            pl.BlockSpec((TM, D_IN), lambda i: (i, 0)),
            pl.BlockSpec((D_IN, D_EMB), lambda i: (0, 0)),
            pl.BlockSpec((D_EMB, V), lambda i: (0, 0)),
            pl.BlockSpec((V, D_EMB), lambda i: (0, 0)),
            pl.BlockSpec((D_EMB, D_IN), lambda i: (0, 0)),
        ],
        out_specs=[
            pl.BlockSpec((TM, 1), lambda i: (i, 0)),
            pl.BlockSpec((V, TW), lambda i: (0, 0)),
            pl.BlockSpec((1, 1), lambda i: (0, 0)),
        ],
        out_shape=[
            jax.ShapeDtypeStruct((M, 1), jnp.int32),
            jax.ShapeDtypeStruct((V, TW), jnp.float32),
            jax.ShapeDtypeStruct((1, 1), jnp.float32),
        ],
        scratch_shapes=[
            pltpu.VMEM((D_EMB, V), jnp.float32),
            pltpu.VMEM((8, V), jnp.float32),
        ],
    )(zf, wint, cbt, cb, woutt)


NC, NS = 2, 16
NW = NC * NS
B_PER_W = M // NW
KCH = B_PER_W // 128


def _gather_body(table_hbm, idx_hbm, out_hbm, idx_v, rows_v, sem):
    wid = lax.axis_index("s") * NC + lax.axis_index("c")
    pltpu.sync_copy(idx_hbm.at[wid], idx_v)
    copies = [
        pltpu.async_copy(table_hbm.at[idx_v.at[j]], rows_v.at[j], sem)
        for j in range(KCH)
    ]
    for cp in copies:
        cp.wait()
    pltpu.sync_copy(rows_v, out_hbm.at[wid])


def _run_gather(table, idx3d):
    mesh = plsc.VectorSubcoreMesh(core_axis_name="c", subcore_axis_name="s")
    k = functools.partial(
        pl.kernel,
        mesh=mesh,
        out_type=jax.ShapeDtypeStruct((NW, KCH, 128, TW), jnp.float32),
        scratch_types=[
            pltpu.VMEM((KCH, 128), jnp.int32),
            pltpu.VMEM((KCH, 128, TW), jnp.float32),
            pltpu.SemaphoreType.DMA,
        ],
    )(_gather_body)
    return k(table, idx3d)


def kernel(z, codebook, W_in, W_out):
    zf = z.reshape(M, D_IN)
    idx2, table, loss = _run_argmin(zf, W_in.T, codebook.T, codebook, W_out.T)
    idx3d = idx2.reshape(NW, KCH, 128)
    rows = _run_gather(table, idx3d).reshape(M, TW)
    out = rows[:, :D_IN]
    return (out.reshape(B, N, D_IN),
            idx2.reshape(B, N),
            loss[0, 0])


# speedup vs baseline: 6.5307x; 1.0403x over previous
import functools

import jax
import jax.numpy as jnp
from jax import lax
from jax.experimental import pallas as pl
from jax.experimental.pallas import tpu as pltpu
from jax.experimental.pallas import tpu_sc as plsc

B, N, D_IN = 16, 1024, 64
M = B * N
D_EMB = 32
V = 8192
TW = 128

TM = 1024
SUB = 128
NSUB = V // SUB
GRID = M // TM
VC = 512
LOSS_SCALE = 2.0 / (M * D_EMB)


def _argmin_body(z_ref, wint_ref, cbt_ref, cb_ref, woutt_ref,
                 idx_ref, t_ref, loss_ref, cbt2_ref, cn_ref):
    i = pl.program_id(0)

    @pl.when(i == 0)
    def _():
        loss_ref[...] = jnp.zeros((1, 1), jnp.float32)
        def cbody(c, carry):
            rows = cb_ref[pl.ds(c * VC, VC), :]
            co = jnp.dot(rows, woutt_ref[...])
            t_ref[pl.ds(c * VC, VC), :] = jnp.concatenate(
                [co, jnp.zeros((VC, TW - D_IN), jnp.float32)], axis=1)
            return carry

        lax.fori_loop(0, V // VC, cbody, 0)
        cbt = cbt_ref[...]
        cbt2_ref[...] = cbt + cbt
        cn = jnp.sum(cbt * cbt, axis=0, keepdims=True)
        cn_ref[...] = jnp.broadcast_to(cn, (8, V))

    z = z_ref[...]
    z_in = jnp.dot(z, wint_ref[...])
    zn = jnp.sum(z_in * z_in, axis=1, keepdims=True)
    znb = jnp.broadcast_to(zn, (TM, SUB)).reshape(TM // 8, 8, SUB)

    def body(j, carry):
        mv, mj = carry
        cbt2 = cbt2_ref[:, pl.ds(j * SUB, SUB)]
        cn8 = cn_ref[:, pl.ds(j * SUB, SUB)].reshape(1, 8, SUB)
        zc2 = jnp.dot(z_in, cbt2).reshape(TM // 8, 8, SUB)
        dist = (znb + cn8) - zc2
        better = dist < mv
        return (jnp.where(better, dist, mv),
                jnp.where(better, j, mj))

    mv0 = jnp.full((TM // 8, 8, SUB), jnp.inf, jnp.float32)
    mj0 = jnp.zeros((TM // 8, 8, SUB), jnp.int32)
    mv3, mj3 = lax.fori_loop(0, NSUB, body, (mv0, mj0), unroll=64)
    mv, mj = mv3.reshape(TM, SUB), mj3.reshape(TM, SUB)

    lv = jnp.min(mv, axis=1, keepdims=True)
    lanes = lax.broadcasted_iota(jnp.int32, (TM, SUB), 1)
    cand = jnp.where(mv == lv, mj * SUB + lanes, 2**31 - 1)
    idx_ref[...] = jnp.min(cand, axis=1, keepdims=True)
    loss_ref[...] += jnp.sum(lv).reshape(1, 1)

    @pl.when(i == pl.num_programs(0) - 1)
    def _():
        loss_ref[...] *= LOSS_SCALE


def _run_argmin(zf, wint, cbt, cb, woutt):
    return pl.pallas_call(
        _argmin_body,
        grid=(GRID,),
        in_specs=[
            pl.BlockSpec((TM, D_IN), lambda i: (i, 0)),
            pl.BlockSpec((D_IN, D_EMB), lambda i: (0, 0)),
            pl.BlockSpec((D_EMB, V), lambda i: (0, 0)),
            pl.BlockSpec((V, D_EMB), lambda i: (0, 0)),
            pl.BlockSpec((D_EMB, D_IN), lambda i: (0, 0)),
        ],
        out_specs=[
            pl.BlockSpec((TM, 1), lambda i: (i, 0)),
            pl.BlockSpec((V, TW), lambda i: (0, 0)),
            pl.BlockSpec((1, 1), lambda i: (0, 0)),
        ],
        out_shape=[
            jax.ShapeDtypeStruct((M, 1), jnp.int32),
            jax.ShapeDtypeStruct((V, TW), jnp.float32),
            jax.ShapeDtypeStruct((1, 1), jnp.float32),
        ],
        scratch_shapes=[
            pltpu.VMEM((D_EMB, V), jnp.float32),
            pltpu.VMEM((8, V), jnp.float32),
        ],
    )(zf, wint, cbt, cb, woutt)


NC, NS = 2, 16
NW = NC * NS
B_PER_W = M // NW
KCH = B_PER_W // 128


def _gather_body(table_hbm, idx_hbm, out_hbm, idx_v, rows_v, sem):
    wid = lax.axis_index("s") * NC + lax.axis_index("c")
    pltpu.sync_copy(idx_hbm.at[wid], idx_v)
    copies = [
        pltpu.async_copy(table_hbm.at[idx_v.at[j]], rows_v.at[j], sem)
        for j in range(KCH)
    ]
    for cp in copies:
        cp.wait()
    pltpu.sync_copy(rows_v, out_hbm.at[wid])


def _run_gather(table, idx3d):
    mesh = plsc.VectorSubcoreMesh(core_axis_name="c", subcore_axis_name="s")
    k = functools.partial(
        pl.kernel,
        mesh=mesh,
        out_type=jax.ShapeDtypeStruct((NW, KCH, 128, TW), jnp.float32),
        scratch_types=[
            pltpu.VMEM((KCH, 128), jnp.int32),
            pltpu.VMEM((KCH, 128, TW), jnp.float32),
            pltpu.SemaphoreType.DMA,
        ],
    )(_gather_body)
    return k(table, idx3d)


def kernel(z, codebook, W_in, W_out):
    zf = z.reshape(M, D_IN)
    idx2, table, loss = _run_argmin(zf, W_in.T, codebook.T, codebook, W_out.T)
    idx3d = idx2.reshape(NW, KCH, 128)
    rows = _run_gather(table, idx3d).reshape(M, TW)
    out = rows[:, :D_IN]
    return (out.reshape(B, N, D_IN),
            idx2.reshape(B, N),
            loss[0, 0])
